# baseline (speedup 1.0000x reference)
import functools

import jax
import jax.numpy as jnp
from jax import lax
from jax.experimental import pallas as pl
from jax.experimental.pallas import tpu as pltpu
from jax.experimental.pallas import tpu_sc as plsc

N, E, C, K = 10000, 320000, 128, 27
EPS = 1e-5

NC, NS = 2, 16
EW = E // (NC * NS)
CH = 128
NCH = 80
EWP = NCH * CH
NPH = NCH // 2
NPAD = 10240
DUMMY = N
ZR = NPAD // NS
NBUF = 2

_mesh = plsc.VectorSubcoreMesh(core_axis_name="c", subcore_axis_name="s",
                               num_cores=NC, num_subcores=NS)


@functools.partial(
    pl.kernel,
    out_type=pltpu.HBM((NC * NPAD, C), jnp.float32),
    mesh=_mesh,
    scratch_types=[
        pltpu.VMEM((NPH, CH), jnp.int32),
        pltpu.VMEM((NPH, CH), jnp.int32),
        pltpu.VMEM((CH, C), jnp.float32),
        pltpu.VMEM((CH, C), jnp.float32),
        pltpu.VMEM_SHARED((NPAD, C), jnp.float32),
        pltpu.SemaphoreType.DMA,
        pltpu.SemaphoreType.DMA,
        pltpu.SemaphoreType.DMA,
        pltpu.SemaphoreType.DMA,
    ],
)
def _sc_gather_segsum(gidx_h, sidx_h, xk_h, out_h,
                      gidx, sidx, rows0, rows1, acc,
                      sem0, sem1, ssem0, ssem1):
    c = lax.axis_index("c")
    s = lax.axis_index("s")
    wid = c * NS + s
    rows = (rows0, rows1)
    sems = (sem0, sem1)

    zero16 = jnp.zeros((16,), jnp.float32)

    def _zrow(r, carry):
        for l in range(C // 16):
            rows0[r, pl.ds(l * 16, 16)] = zero16
        return carry

    lax.fori_loop(0, CH, _zrow, 0)
    for z in range(ZR // CH):
        pltpu.sync_copy(rows0, acc.at[pl.ds(s * ZR + z * CH, CH)])
    plsc.subcore_barrier()

    ssems = (ssem0, ssem1)

    def _start(b, j):
        pltpu.async_copy(xk_h.at[gidx.at[j]], rows[b], sems[b])

    def _wait(b):
        pltpu.make_async_copy(xk_h.at[gidx.at[0]], rows[b], sems[b]).wait()

    def _scat(b, j):
        pltpu.async_copy(rows[b], acc.at[sidx.at[j]], ssems[b], add=True)

    def _swait(b):
        pltpu.make_async_copy(rows[b], acc.at[sidx.at[0]], ssems[b]).wait()

    for c0 in (0, NPH):
        pltpu.sync_copy(gidx_h.at[wid, pl.ds(c0, NPH)], gidx)
        pltpu.sync_copy(sidx_h.at[wid, pl.ds(c0, NPH)], sidx)
        for b in range(NBUF):
            _start(b, b)

        def _step(t, carry):
            base = t * NBUF
            for b in range(NBUF):
                j = base + b
                _wait(b)
                _scat(b, j)

            for b in range(NBUF):
                j = base + b
                _swait(b)
                _start(b, j + NBUF)

            return carry

        lax.fori_loop(0, (NPH - NBUF) // NBUF, _step, 0)
        for j in range(NPH - NBUF, NPH):
            b = j % NBUF
            _wait(b)
            _scat(b, j)
        for j in range(NPH - NBUF, NPH):
            _swait(j % NBUF)

    plsc.subcore_barrier()
    pltpu.sync_copy(acc.at[pl.ds(s * ZR, ZR)],
                    out_h.at[pl.ds(c * NPAD + s * ZR, ZR)])


BN1 = 1000
NB = N // BN1


def _mm1_body(x_ref, w_ref, o_ref):
    o_ref[0] = jnp.dot(x_ref[...], w_ref[0], preferred_element_type=jnp.float32)


def _einsum_xw(xin, W):
    return pl.pallas_call(
        _mm1_body,
        grid=(NB, K),
        in_specs=[pl.BlockSpec((BN1, C), lambda nb, k: (nb, 0)),
                  pl.BlockSpec((1, C, C), lambda nb, k: (k, 0, 0))],
        out_specs=pl.BlockSpec((1, BN1, C), lambda nb, k: (k, nb, 0)),
        out_shape=jax.ShapeDtypeStruct((K, N, C), jnp.float32),
    )(xin, W)


_pblk0 = pl.BlockSpec((1, BN1, C), lambda i, *_: (0, i, 0))
_pblk1 = pl.BlockSpec((1, BN1, C), lambda i, *_: (1, i, 0))


def _stats_body(pa_ref, pb_ref, s_ref, q_ref):
    y = pa_ref[0] + pb_ref[0]

    @pl.when(pl.program_id(0) == 0)
    def _():
        s_ref[...] = jnp.zeros_like(s_ref)
        q_ref[...] = jnp.zeros_like(q_ref)

    s_ref[...] += jnp.sum(y, axis=0, keepdims=True)
    q_ref[...] += jnp.sum(y * y, axis=0, keepdims=True)


def _stats(p):
    return pl.pallas_call(
        _stats_body,
        grid=(NB,),
        in_specs=[_pblk0, _pblk1],
        out_specs=[pl.BlockSpec((1, C), lambda i: (0, 0)),
                   pl.BlockSpec((1, C), lambda i: (0, 0))],
        out_shape=[jax.ShapeDtypeStruct((1, C), jnp.float32),
                   jax.ShapeDtypeStruct((1, C), jnp.float32)],
    )(p, p)


def _mm2_body(pa_ref, pb_ref, s_ref, q_ref, g_ref, b_ref, w_ref, o_ref):
    mu = s_ref[0] * (1.0 / N)
    var = q_ref[0] * (1.0 / N) - mu * mu
    inv = lax.rsqrt(var + EPS) * g_ref[0]
    yn = jnp.maximum((pa_ref[0] + pb_ref[0] - mu) * inv + b_ref[0], 0.0)
    o_ref[0] = jnp.dot(yn, w_ref[0], preferred_element_type=jnp.float32)


def _einsum_bn_relu(p, ssum, sq, gamma, beta, W):
    vec = pl.BlockSpec((1, C), lambda nb, k: (0, 0))
    return pl.pallas_call(
        _mm2_body,
        grid=(NB, K),
        in_specs=[_pblk0, _pblk1, vec, vec, vec, vec,
                  pl.BlockSpec((1, C, C), lambda nb, k: (k, 0, 0))],
        out_specs=pl.BlockSpec((1, BN1, C), lambda nb, k: (k, nb, 0)),
        out_shape=jax.ShapeDtypeStruct((K, N, C), jnp.float32),
    )(p, p, ssum, sq, gamma, beta, W)


def _fin_body(pa_ref, pb_ref, s_ref, q_ref, g_ref, b_ref, x_ref, o_ref):
    mu = s_ref[0] * (1.0 / N)
    var = q_ref[0] * (1.0 / N) - mu * mu
    inv = lax.rsqrt(var + EPS) * g_ref[0]
    o_ref[...] = jnp.maximum(
        (pa_ref[0] + pb_ref[0] - mu) * inv + b_ref[0] + x_ref[...], 0.0)


def _final(p, ssum, sq, gamma, beta, x):
    blk = pl.BlockSpec((BN1, C), lambda i: (i, 0))
    vec = pl.BlockSpec((1, C), lambda i: (0, 0))
    return pl.pallas_call(
        _fin_body,
        grid=(NB,),
        in_specs=[_pblk0, _pblk1, vec, vec, vec, vec, blk],
        out_specs=blk,
        out_shape=jax.ShapeDtypeStruct((N, C), jnp.float32),
    )(p, p, ssum, sq, gamma, beta, x)


def kernel(x, edge_index, kernel_offset, W1, gamma1, beta1, W2, gamma2, beta2):
    src = edge_index[0]
    dst = edge_index[1]
    off = kernel_offset
    g1 = gamma1.reshape(1, C)
    b1 = beta1.reshape(1, C)
    g2 = gamma2.reshape(1, C)
    b2 = beta2.reshape(1, C)

    gflat = (off * N + src).reshape(NC * NS, EW)
    sflat = dst.reshape(NC * NS, EW)
    gidx = jnp.pad(gflat, ((0, 0), (0, EWP - EW))).reshape(NC * NS, NCH, CH)
    sidx = jnp.pad(sflat, ((0, 0), (0, EWP - EW)),
                   constant_values=DUMMY).reshape(NC * NS, NCH, CH)
    gidx_r = jax.new_ref(gidx, memory_space=pltpu.MemorySpace.HBM)
    sidx_r = jax.new_ref(sidx, memory_space=pltpu.MemorySpace.HBM)

    xk1 = _einsum_xw(x, W1).reshape(K * N, C)
    xk1_r = jax.new_ref(xk1, memory_space=pltpu.MemorySpace.HBM)
    p1 = _sc_gather_segsum(gidx_r, sidx_r, xk1_r).reshape(NC, NPAD, C)
    s1, q1 = _stats(p1)
    xk2 = _einsum_bn_relu(p1, s1, q1, g1, b1, W2).reshape(K * N, C)
    xk2_r = jax.new_ref(xk2, memory_space=pltpu.MemorySpace.HBM)
    p2 = _sc_gather_segsum(gidx_r, sidx_r, xk2_r).reshape(NC, NPAD, C)
    s2, q2 = _stats(p2)
    return _final(p2, s2, q2, g2, b2, x)


# speedup vs baseline: 1.0561x; 1.0561x over previous
import functools

import jax
import jax.numpy as jnp
from jax import lax
from jax.experimental import pallas as pl
from jax.experimental.pallas import tpu as pltpu
from jax.experimental.pallas import tpu_sc as plsc

N, E, C, K = 10000, 320000, 128, 27
EPS = 1e-5

NC, NS = 2, 16
EW = E // (NC * NS)
CH = 128
NCH = 80
EWP = NCH * CH
NPH = NCH // 2
NPAD = 10240
DUMMY = N
ZR = NPAD // NS
NBUF = 2

_mesh = plsc.VectorSubcoreMesh(core_axis_name="c", subcore_axis_name="s",
                               num_cores=NC, num_subcores=NS)


@functools.partial(
    pl.kernel,
    out_type=pltpu.HBM((NC * NPAD, C), jnp.float32),
    mesh=_mesh,
    scratch_types=[
        pltpu.VMEM((NPH, CH), jnp.int32),
        pltpu.VMEM((NPH, CH), jnp.int32),
        pltpu.VMEM((CH, C), jnp.float32),
        pltpu.VMEM((CH, C), jnp.float32),
        pltpu.VMEM_SHARED((NPAD, C), jnp.float32),
        pltpu.SemaphoreType.DMA,
        pltpu.SemaphoreType.DMA,
    ],
)
def _sc_gather_segsum(gidx_h, sidx_h, xk_h, out_h,
                      gidx, sidx, rows0, rows1, acc, sem0, sem1):
    c = lax.axis_index("c")
    s = lax.axis_index("s")
    wid = c * NS + s
    rows = (rows0, rows1)
    sems = (sem0, sem1)

    zero16 = jnp.zeros((16,), jnp.float32)

    def _zrow(r, carry):
        for l in range(C // 16):
            rows0[r, pl.ds(l * 16, 16)] = zero16
        return carry

    lax.fori_loop(0, CH, _zrow, 0)
    for z in range(ZR // CH):
        pltpu.sync_copy(rows0, acc.at[pl.ds(s * ZR + z * CH, CH)])
    plsc.subcore_barrier()

    def _start(b, j):
        pltpu.async_copy(xk_h.at[gidx.at[j]], rows[b], sems[b])

    def _wait(b):
        pltpu.make_async_copy(xk_h.at[gidx.at[0]], rows[b], sems[b]).wait()

    def _scat(b, j):
        pltpu.sync_copy(rows[b], acc.at[sidx.at[j]], add=True)

    for c0 in (0, NPH):
        pltpu.sync_copy(gidx_h.at[wid, pl.ds(c0, NPH)], gidx)
        pltpu.sync_copy(sidx_h.at[wid, pl.ds(c0, NPH)], sidx)
        for b in range(NBUF):
            _start(b, b)

        def _step(t, carry):
            base = t * NBUF
            for b in range(NBUF):
                j = base + b
                _wait(b)
                _scat(b, j)
                _start(b, j + NBUF)
            return carry

        lax.fori_loop(0, (NPH - NBUF) // NBUF, _step, 0)
        for j in range(NPH - NBUF, NPH):
            b = j % NBUF
            _wait(b)
            _scat(b, j)

    plsc.subcore_barrier()
    pltpu.sync_copy(acc.at[pl.ds(s * ZR, ZR)],
                    out_h.at[pl.ds(c * NPAD + s * ZR, ZR)])


BN1 = 1000
NB = N // BN1


def _mm1_body(x_ref, w_ref, o_ref):
    o_ref[0] = jnp.dot(x_ref[...], w_ref[0], preferred_element_type=jnp.float32)


def _einsum_xw(xin, W):
    return pl.pallas_call(
        _mm1_body,
        grid=(NB, K),
        in_specs=[pl.BlockSpec((BN1, C), lambda nb, k: (nb, 0)),
                  pl.BlockSpec((1, C, C), lambda nb, k: (k, 0, 0))],
        out_specs=pl.BlockSpec((1, BN1, C), lambda nb, k: (k, nb, 0)),
        out_shape=jax.ShapeDtypeStruct((K, N, C), jnp.float32),
    )(xin, W)


_pblk0 = pl.BlockSpec((1, BN1, C), lambda i, *_: (0, i, 0))
_pblk1 = pl.BlockSpec((1, BN1, C), lambda i, *_: (1, i, 0))


def _stats_body(pa_ref, pb_ref, s_ref, q_ref):
    y = pa_ref[0] + pb_ref[0]

    @pl.when(pl.program_id(0) == 0)
    def _():
        s_ref[...] = jnp.zeros_like(s_ref)
        q_ref[...] = jnp.zeros_like(q_ref)

    s_ref[...] += jnp.sum(y, axis=0, keepdims=True)
    q_ref[...] += jnp.sum(y * y, axis=0, keepdims=True)


def _stats(p):
    return pl.pallas_call(
        _stats_body,
        grid=(NB,),
        in_specs=[_pblk0, _pblk1],
        out_specs=[pl.BlockSpec((1, C), lambda i: (0, 0)),
                   pl.BlockSpec((1, C), lambda i: (0, 0))],
        out_shape=[jax.ShapeDtypeStruct((1, C), jnp.float32),
                   jax.ShapeDtypeStruct((1, C), jnp.float32)],
    )(p, p)


def _mm2_body(pa_ref, pb_ref, s_ref, q_ref, g_ref, b_ref, w_ref, o_ref):
    mu = s_ref[0] * (1.0 / N)
    var = q_ref[0] * (1.0 / N) - mu * mu
    inv = lax.rsqrt(var + EPS) * g_ref[0]
    yn = jnp.maximum((pa_ref[0] + pb_ref[0] - mu) * inv + b_ref[0], 0.0)
    o_ref[0] = jnp.dot(yn, w_ref[0], preferred_element_type=jnp.float32)


def _einsum_bn_relu(p, ssum, sq, gamma, beta, W):
    vec = pl.BlockSpec((1, C), lambda nb, k: (0, 0))
    return pl.pallas_call(
        _mm2_body,
        grid=(NB, K),
        in_specs=[_pblk0, _pblk1, vec, vec, vec, vec,
                  pl.BlockSpec((1, C, C), lambda nb, k: (k, 0, 0))],
        out_specs=pl.BlockSpec((1, BN1, C), lambda nb, k: (k, nb, 0)),
        out_shape=jax.ShapeDtypeStruct((K, N, C), jnp.float32),
    )(p, p, ssum, sq, gamma, beta, W)


def _fin_body(pa_ref, pb_ref, s_ref, q_ref, g_ref, b_ref, x_ref, o_ref):
    mu = s_ref[0] * (1.0 / N)
    var = q_ref[0] * (1.0 / N) - mu * mu
    inv = lax.rsqrt(var + EPS) * g_ref[0]
    o_ref[...] = jnp.maximum(
        (pa_ref[0] + pb_ref[0] - mu) * inv + b_ref[0] + x_ref[...], 0.0)


def _final(p, ssum, sq, gamma, beta, x):
    blk = pl.BlockSpec((BN1, C), lambda i: (i, 0))
    vec = pl.BlockSpec((1, C), lambda i: (0, 0))
    return pl.pallas_call(
        _fin_body,
        grid=(NB,),
        in_specs=[_pblk0, _pblk1, vec, vec, vec, vec, blk],
        out_specs=blk,
        out_shape=jax.ShapeDtypeStruct((N, C), jnp.float32),
    )(p, p, ssum, sq, gamma, beta, x)


def kernel(x, edge_index, kernel_offset, W1, gamma1, beta1, W2, gamma2, beta2):
    src = edge_index[0]
    dst = edge_index[1]
    off = kernel_offset
    g1 = gamma1.reshape(1, C)
    b1 = beta1.reshape(1, C)
    g2 = gamma2.reshape(1, C)
    b2 = beta2.reshape(1, C)

    gflat = (off * N + src).reshape(NC * NS, EW)
    sflat = dst.reshape(NC * NS, EW)
    gidx = jnp.pad(gflat, ((0, 0), (0, EWP - EW))).reshape(NC * NS, NCH, CH)
    sidx = jnp.pad(sflat, ((0, 0), (0, EWP - EW)),
                   constant_values=DUMMY).reshape(NC * NS, NCH, CH)
    gidx_r = jax.new_ref(gidx, memory_space=pltpu.MemorySpace.HBM)
    sidx_r = jax.new_ref(sidx, memory_space=pltpu.MemorySpace.HBM)

    xk1 = _einsum_xw(x, W1).reshape(K * N, C)
    xk1_r = jax.new_ref(xk1, memory_space=pltpu.MemorySpace.HBM)
    p1 = _sc_gather_segsum(gidx_r, sidx_r, xk1_r).reshape(NC, NPAD, C)
    s1, q1 = _stats(p1)
    xk2 = _einsum_bn_relu(p1, s1, q1, g1, b1, W2).reshape(K * N, C)
    xk2_r = jax.new_ref(xk2, memory_space=pltpu.MemorySpace.HBM)
    p2 = _sc_gather_segsum(gidx_r, sidx_r, xk2_r).reshape(NC, NPAD, C)
    s2, q2 = _stats(p2)
    return _final(p2, s2, q2, g2, b2, x)


# speedup vs baseline: 1.2234x; 1.1583x over previous
import functools

import jax
import jax.numpy as jnp
from jax import lax
from jax.experimental import pallas as pl
from jax.experimental.pallas import tpu as pltpu
from jax.experimental.pallas import tpu_sc as plsc

N, E, C, K = 10000, 320000, 128, 27
EPS = 1e-5

NC, NS = 2, 16
EW = E // (NC * NS)
CH = 128
NCH = 80
EWP = NCH * CH
NPH = NCH // 2
NPAD = 10240
DUMMY = N
ZR = NPAD // NS
NBUF = 2

_mesh = plsc.VectorSubcoreMesh(core_axis_name="c", subcore_axis_name="s",
                               num_cores=NC, num_subcores=NS)


@functools.partial(
    pl.kernel,
    out_type=pltpu.HBM((NC * NPAD, C), jnp.float32),
    mesh=_mesh,
    scratch_types=[
        pltpu.VMEM((NPH, CH), jnp.int32),
        pltpu.VMEM((NPH, CH), jnp.int32),
        pltpu.VMEM((CH, C), jnp.float32),
        pltpu.VMEM((CH, C), jnp.float32),
        pltpu.VMEM_SHARED((NPAD, C), jnp.float32),
        pltpu.SemaphoreType.DMA,
        pltpu.SemaphoreType.DMA,
    ],
)
def _sc_gather_segsum(gidx_h, sidx_h, xk_h, out_h,
                      gidx, sidx, rows0, rows1, acc, sem0, sem1):
    c = lax.axis_index("c")
    s = lax.axis_index("s")
    wid = c * NS + s
    rows = (rows0, rows1)
    sems = (sem0, sem1)

    zero16 = jnp.zeros((16,), jnp.float32)

    def _zrow(r, carry):
        for l in range(C // 16):
            rows0[r, pl.ds(l * 16, 16)] = zero16
        return carry

    lax.fori_loop(0, CH, _zrow, 0)
    for z in range(ZR // CH):
        pltpu.sync_copy(rows0, acc.at[pl.ds(s * ZR + z * CH, CH)])
    plsc.subcore_barrier()

    def _start(b, j):
        pltpu.async_copy(xk_h.at[gidx.at[j]], rows[b], sems[b])

    def _wait(b):
        pltpu.make_async_copy(xk_h.at[gidx.at[0]], rows[b], sems[b]).wait()

    def _scat(b, j):
        pltpu.sync_copy(rows[b], acc.at[sidx.at[j]], add=True)

    for c0 in (0, NPH):
        pltpu.sync_copy(gidx_h.at[wid, pl.ds(c0, NPH)], gidx)
        pltpu.sync_copy(sidx_h.at[wid, pl.ds(c0, NPH)], sidx)
        for b in range(NBUF):
            _start(b, b)

        def _step(t, carry):
            base = t * NBUF
            for b in range(NBUF):
                j = base + b
                _wait(b)
                _scat(b, j)
                _start(b, j + NBUF)
            return carry

        lax.fori_loop(0, (NPH - NBUF) // NBUF, _step, 0)
        for j in range(NPH - NBUF, NPH):
            b = j % NBUF
            _wait(b)
            _scat(b, j)

    plsc.subcore_barrier()
    pltpu.sync_copy(acc.at[pl.ds(s * ZR, ZR)],
                    out_h.at[pl.ds(c * NPAD + s * ZR, ZR)])


BN1 = 2000
NB = N // BN1


def _mm1_body(x_ref, w_ref, o_ref):
    o_ref[0] = jnp.dot(x_ref[...], w_ref[0], preferred_element_type=jnp.float32)


def _einsum_xw(xin, W):
    return pl.pallas_call(
        _mm1_body,
        grid=(NB, K),
        in_specs=[pl.BlockSpec((BN1, C), lambda nb, k: (nb, 0)),
                  pl.BlockSpec((1, C, C), lambda nb, k: (k, 0, 0))],
        out_specs=pl.BlockSpec((1, BN1, C), lambda nb, k: (k, nb, 0)),
        out_shape=jax.ShapeDtypeStruct((K, N, C), jnp.float32),
    )(xin, W)


_pblk0 = pl.BlockSpec((1, BN1, C), lambda i, *_: (0, i, 0))
_pblk1 = pl.BlockSpec((1, BN1, C), lambda i, *_: (1, i, 0))


def _stats_body(pa_ref, pb_ref, s_ref, q_ref):
    y = pa_ref[0] + pb_ref[0]

    @pl.when(pl.program_id(0) == 0)
    def _():
        s_ref[...] = jnp.zeros_like(s_ref)
        q_ref[...] = jnp.zeros_like(q_ref)

    s_ref[...] += jnp.sum(y, axis=0, keepdims=True)
    q_ref[...] += jnp.sum(y * y, axis=0, keepdims=True)


def _stats(p):
    return pl.pallas_call(
        _stats_body,
        grid=(NB,),
        in_specs=[_pblk0, _pblk1],
        out_specs=[pl.BlockSpec((1, C), lambda i: (0, 0)),
                   pl.BlockSpec((1, C), lambda i: (0, 0))],
        out_shape=[jax.ShapeDtypeStruct((1, C), jnp.float32),
                   jax.ShapeDtypeStruct((1, C), jnp.float32)],
    )(p, p)


def _mm2_body(pa_ref, pb_ref, s_ref, q_ref, g_ref, b_ref, w_ref, o_ref):
    mu = s_ref[0] * (1.0 / N)
    var = q_ref[0] * (1.0 / N) - mu * mu
    inv = lax.rsqrt(var + EPS) * g_ref[0]
    yn = jnp.maximum((pa_ref[0] + pb_ref[0] - mu) * inv + b_ref[0], 0.0)
    o_ref[0] = jnp.dot(yn, w_ref[0], preferred_element_type=jnp.float32)


def _einsum_bn_relu(p, ssum, sq, gamma, beta, W):
    vec = pl.BlockSpec((1, C), lambda nb, k: (0, 0))
    return pl.pallas_call(
        _mm2_body,
        grid=(NB, K),
        in_specs=[_pblk0, _pblk1, vec, vec, vec, vec,
                  pl.BlockSpec((1, C, C), lambda nb, k: (k, 0, 0))],
        out_specs=pl.BlockSpec((1, BN1, C), lambda nb, k: (k, nb, 0)),
        out_shape=jax.ShapeDtypeStruct((K, N, C), jnp.float32),
    )(p, p, ssum, sq, gamma, beta, W)


def _fin_body(pa_ref, pb_ref, s_ref, q_ref, g_ref, b_ref, x_ref, o_ref):
    mu = s_ref[0] * (1.0 / N)
    var = q_ref[0] * (1.0 / N) - mu * mu
    inv = lax.rsqrt(var + EPS) * g_ref[0]
    o_ref[...] = jnp.maximum(
        (pa_ref[0] + pb_ref[0] - mu) * inv + b_ref[0] + x_ref[...], 0.0)


def _final(p, ssum, sq, gamma, beta, x):
    blk = pl.BlockSpec((BN1, C), lambda i: (i, 0))
    vec = pl.BlockSpec((1, C), lambda i: (0, 0))
    return pl.pallas_call(
        _fin_body,
        grid=(NB,),
        in_specs=[_pblk0, _pblk1, vec, vec, vec, vec, blk],
        out_specs=blk,
        out_shape=jax.ShapeDtypeStruct((N, C), jnp.float32),
    )(p, p, ssum, sq, gamma, beta, x)


def kernel(x, edge_index, kernel_offset, W1, gamma1, beta1, W2, gamma2, beta2):
    src = edge_index[0]
    dst = edge_index[1]
    off = kernel_offset
    g1 = gamma1.reshape(1, C)
    b1 = beta1.reshape(1, C)
    g2 = gamma2.reshape(1, C)
    b2 = beta2.reshape(1, C)

    gflat = (off * N + src).reshape(NC * NS, EW)
    sflat = dst.reshape(NC * NS, EW)
    gidx = jnp.pad(gflat, ((0, 0), (0, EWP - EW))).reshape(NC * NS, NCH, CH)
    sidx = jnp.pad(sflat, ((0, 0), (0, EWP - EW)),
                   constant_values=DUMMY).reshape(NC * NS, NCH, CH)
    gidx_r = jax.new_ref(gidx, memory_space=pltpu.MemorySpace.HBM)
    sidx_r = jax.new_ref(sidx, memory_space=pltpu.MemorySpace.HBM)

    xk1 = _einsum_xw(x, W1).reshape(K * N, C)
    xk1_r = jax.new_ref(xk1, memory_space=pltpu.MemorySpace.HBM)
    p1 = _sc_gather_segsum(gidx_r, sidx_r, xk1_r).reshape(NC, NPAD, C)
    s1, q1 = _stats(p1)
    xk2 = _einsum_bn_relu(p1, s1, q1, g1, b1, W2).reshape(K * N, C)
    xk2_r = jax.new_ref(xk2, memory_space=pltpu.MemorySpace.HBM)
    p2 = _sc_gather_segsum(gidx_r, sidx_r, xk2_r).reshape(NC, NPAD, C)
    s2, q2 = _stats(p2)
    return _final(p2, s2, q2, g2, b2, x)
